# baseline (speedup 1.0000x reference)
import functools

import jax
import jax.numpy as jnp
from jax import lax
from jax.experimental import pallas as pl
from jax.experimental.pallas import tpu as pltpu
from jax.experimental.pallas import tpu_sc as plsc

_B = 16384
_D = 33
_NC = 2
_NS = 16
_NW = _NC * _NS
_BPW = _B // _NW
_L = 16
_NGROUP = _BPW // _L
_DP = 48


_NQ = 4
_GPQ = _NGROUP // _NQ
_RPQ = _BPW // _NQ


def _sc_body(
    ids_hbm, glog_hbm, table_hbm, out_hbm, ids_v, rows_v, outt_v, glog_v, *sems
):
    wid = lax.axis_index("s") * _NC + lax.axis_index("c")
    base = wid * _BPW

    pltpu.sync_copy(glog_hbm, glog_v.at[pl.ds(0, _D)])
    pltpu.sync_copy(ids_hbm.at[pl.ds(base, _BPW)], ids_v)

    for q in range(_NQ):
        sem = sems[q]

        def enqueue(g, carry, sem=sem):
            vid = ids_v[pl.ds(g * _L, _L)]
            for k in range(_L):
                sid = vid[k]
                pltpu.async_copy(
                    table_hbm.at[sid], rows_v.at[g * _L + k, pl.ds(0, _D)], sem
                )
            return carry

        lax.fori_loop(q * _GPQ, (q + 1) * _GPQ, enqueue, 0)

    g0 = glog_v[pl.ds(0, _L)]
    g1 = glog_v[pl.ds(_L, _L)]
    g2 = glog_v[pl.ds(2 * _L, _L)]
    glog_s = [g0[j] for j in range(_L)] + [g1[j] for j in range(_L)] + [g2[0]]

    col_idx = [jnp.full((_L,), j, jnp.int32) for j in range(_D)]

    for q in range(_NQ):
        sem = sems[q]

        def drain(r, carry, sem=sem):
            pltpu.make_async_copy(
                table_hbm.at[0], rows_v.at[0, pl.ds(0, _D)], sem
            ).wait()
            return carry

        lax.fori_loop(0, _RPQ, drain, 0)

        @plsc.parallel_loop(q * _GPQ, (q + 1) * _GPQ)
        def group_body(g):
            row_idx = g * _L + lax.iota(jnp.int32, 16)
            fidx = [[row_idx, col_idx[j]] for j in range(_D)]
            es = [
                jnp.exp(plsc.load_gather(rows_v, fidx[j]) + glog_s[j])
                for j in range(_D)
            ]
            acc = list(es)
            while len(acc) > 1:
                nxt = [acc[i] + acc[i + 1] for i in range(0, len(acc) - 1, 2)]
                if len(acc) % 2:
                    nxt.append(acc[-1])
                acc = nxt
            inv = 1.0 / acc[0]
            for j in range(_D):
                outt_v[j, pl.ds(g * _L, _L)] = es[j] * inv

    pltpu.sync_copy(outt_v, out_hbm.at[pl.ds(0, _D), pl.ds(base, _BPW)])


@functools.partial(
    pl.kernel,
    out_type=jax.ShapeDtypeStruct((_D, _B), jnp.float32),
    mesh=plsc.VectorSubcoreMesh(core_axis_name="c", subcore_axis_name="s"),
    scratch_types=[
        pltpu.VMEM((_BPW,), jnp.int32),
        pltpu.VMEM((_BPW, _DP), jnp.float32),
        pltpu.VMEM((_D, _BPW), jnp.float32),
        pltpu.VMEM((3 * _L,), jnp.float32),
        pltpu.SemaphoreType.DMA,
        pltpu.SemaphoreType.DMA,
        pltpu.SemaphoreType.DMA,
        pltpu.SemaphoreType.DMA,
    ],
    compiler_params=pltpu.CompilerParams(
        needs_layout_passes=False, use_tc_tiling_on_sc=True
    ),
)
def _mixer(
    ids_hbm, glog_hbm, table_hbm, out_hbm, ids_v, rows_v, outt_v, glog_v, *sems
):
    _sc_body(
        ids_hbm, glog_hbm, table_hbm, out_hbm, ids_v, rows_v, outt_v, glog_v, *sems
    )


def kernel(subject_ids, global_logits, subject_bias):
    out = _mixer(subject_ids, global_logits, subject_bias)
    return out.T


# speedup vs baseline: 1.0197x; 1.0197x over previous
import functools

import jax
import jax.numpy as jnp
from jax import lax
from jax.experimental import pallas as pl
from jax.experimental.pallas import tpu as pltpu
from jax.experimental.pallas import tpu_sc as plsc

_B = 16384
_D = 33
_NC = 2
_NS = 16
_NW = _NC * _NS
_BPW = _B // _NW
_L = 16
_NGROUP = _BPW // _L
_DP = 48


_NQ = 1
_GPQ = _NGROUP // _NQ
_RPQ = _BPW // _NQ


def _sc_body(
    ids_hbm, glog_hbm, table_hbm, out_hbm, ids_v, rows_v, outt_v, glog_v, *sems
):
    wid = lax.axis_index("s") * _NC + lax.axis_index("c")
    base = wid * _BPW

    pltpu.sync_copy(glog_hbm, glog_v.at[pl.ds(0, _D)])
    pltpu.sync_copy(ids_hbm.at[pl.ds(base, _BPW)], ids_v)

    for q in range(_NQ):
        sem = sems[q]

        def enqueue(g, carry, sem=sem):
            vid = ids_v[pl.ds(g * _L, _L)]
            for k in range(_L):
                sid = vid[k]
                pltpu.async_copy(
                    table_hbm.at[sid], rows_v.at[g * _L + k, pl.ds(0, _D)], sem
                )
            return carry

        lax.fori_loop(q * _GPQ, (q + 1) * _GPQ, enqueue, 0)

    g0 = glog_v[pl.ds(0, _L)]
    g1 = glog_v[pl.ds(_L, _L)]
    g2 = glog_v[pl.ds(2 * _L, _L)]
    glog_s = [g0[j] for j in range(_L)] + [g1[j] for j in range(_L)] + [g2[0]]

    col_idx = [jnp.full((_L,), j, jnp.int32) for j in range(_D)]

    for q in range(_NQ):
        sem = sems[q]

        def drain(r, carry, sem=sem):
            pltpu.make_async_copy(
                table_hbm.at[0], rows_v.at[0, pl.ds(0, _D)], sem
            ).wait()
            return carry

        lax.fori_loop(0, _RPQ, drain, 0)

        @plsc.parallel_loop(q * _GPQ, (q + 1) * _GPQ)
        def group_body(g):
            row_idx = g * _L + lax.iota(jnp.int32, 16)
            fidx = [[row_idx, col_idx[j]] for j in range(_D)]
            es = [
                jnp.exp(plsc.load_gather(rows_v, fidx[j]) + glog_s[j])
                for j in range(_D)
            ]
            acc = list(es)
            while len(acc) > 1:
                nxt = [acc[i] + acc[i + 1] for i in range(0, len(acc) - 1, 2)]
                if len(acc) % 2:
                    nxt.append(acc[-1])
                acc = nxt
            inv = 1.0 / acc[0]
            for j in range(_D):
                outt_v[j, pl.ds(g * _L, _L)] = es[j] * inv

    pltpu.sync_copy(outt_v, out_hbm.at[pl.ds(0, _D), pl.ds(base, _BPW)])


@functools.partial(
    pl.kernel,
    out_type=jax.ShapeDtypeStruct((_D, _B), jnp.float32),
    mesh=plsc.VectorSubcoreMesh(core_axis_name="c", subcore_axis_name="s"),
    scratch_types=[
        pltpu.VMEM((_BPW,), jnp.int32),
        pltpu.VMEM((_BPW, _DP), jnp.float32),
        pltpu.VMEM((_D, _BPW), jnp.float32),
        pltpu.VMEM((3 * _L,), jnp.float32),
        pltpu.SemaphoreType.DMA,
    ],
    compiler_params=pltpu.CompilerParams(
        needs_layout_passes=False,
        use_tc_tiling_on_sc=True,
        skip_device_barrier=True,
    ),
)
def _mixer(
    ids_hbm, glog_hbm, table_hbm, out_hbm, ids_v, rows_v, outt_v, glog_v, *sems
):
    _sc_body(
        ids_hbm, glog_hbm, table_hbm, out_hbm, ids_v, rows_v, outt_v, glog_v, *sems
    )


def kernel(subject_ids, global_logits, subject_bias):
    out = _mixer(subject_ids, global_logits, subject_bias)
    return out.T


# speedup vs baseline: 1.0387x; 1.0186x over previous
import functools

import jax
import jax.numpy as jnp
from jax import lax
from jax.experimental import pallas as pl
from jax.experimental.pallas import tpu as pltpu
from jax.experimental.pallas import tpu_sc as plsc

_B = 16384
_D = 33
_NC = 2
_NS = 16
_NW = _NC * _NS
_BPW = _B // _NW
_L = 16
_NGROUP = _BPW // _L
_DP = 48


_NQ = 1
_GPQ = _NGROUP // _NQ
_RPQ = _BPW // _NQ


def _sc_body(
    ids_hbm, glog_hbm, table_hbm, out_hbm, ids_v, rows_v, outt_v, glog_v, *sems
):
    wid = lax.axis_index("s") * _NC + lax.axis_index("c")
    base = wid * _BPW

    pltpu.sync_copy(glog_hbm, glog_v.at[pl.ds(0, _D)])
    pltpu.sync_copy(ids_hbm.at[pl.ds(base, _BPW)], ids_v)

    for q in range(_NQ):
        sem = sems[q]

        def enqueue(g, carry, sem=sem):
            vid = ids_v[pl.ds(g * _L, _L)]
            for k in range(_L):
                sid = vid[k]
                pltpu.async_copy(
                    table_hbm.at[sid], rows_v.at[g * _L + k, pl.ds(0, _D)], sem
                )
            return carry

        lax.fori_loop(q * _GPQ, (q + 1) * _GPQ, enqueue, 0)

    g0 = glog_v[pl.ds(0, _L)]
    g1 = glog_v[pl.ds(_L, _L)]
    g2 = glog_v[pl.ds(2 * _L, _L)]
    glog_s = [g0[j] for j in range(_L)] + [g1[j] for j in range(_L)] + [g2[0]]

    col_idx = [jnp.full((_L,), j, jnp.int32) for j in range(_D)]

    for q in range(_NQ):
        sem = sems[q]

        def drain(r, carry, sem=sem):
            for _ in range(8):
                pltpu.make_async_copy(
                    table_hbm.at[0], rows_v.at[0, pl.ds(0, _D)], sem
                ).wait()
            return carry

        lax.fori_loop(0, _RPQ // 8, drain, 0)

        @plsc.parallel_loop(q * _GPQ, (q + 1) * _GPQ)
        def group_body(g):
            row_idx = g * _L + lax.iota(jnp.int32, 16)
            fidx = [[row_idx, col_idx[j]] for j in range(_D)]
            es = [
                jnp.exp(plsc.load_gather(rows_v, fidx[j]) + glog_s[j])
                for j in range(_D)
            ]
            acc = list(es)
            while len(acc) > 1:
                nxt = [acc[i] + acc[i + 1] for i in range(0, len(acc) - 1, 2)]
                if len(acc) % 2:
                    nxt.append(acc[-1])
                acc = nxt
            inv = 1.0 / acc[0]
            for j in range(_D):
                outt_v[j, pl.ds(g * _L, _L)] = es[j] * inv

    pltpu.sync_copy(outt_v, out_hbm.at[pl.ds(0, _D), pl.ds(base, _BPW)])


@functools.partial(
    pl.kernel,
    out_type=jax.ShapeDtypeStruct((_D, _B), jnp.float32),
    mesh=plsc.VectorSubcoreMesh(core_axis_name="c", subcore_axis_name="s"),
    scratch_types=[
        pltpu.VMEM((_BPW,), jnp.int32),
        pltpu.VMEM((_BPW, _DP), jnp.float32),
        pltpu.VMEM((_D, _BPW), jnp.float32),
        pltpu.VMEM((3 * _L,), jnp.float32),
        pltpu.SemaphoreType.DMA,
    ],
    compiler_params=pltpu.CompilerParams(
        needs_layout_passes=False,
        use_tc_tiling_on_sc=True,
        skip_device_barrier=True,
    ),
)
def _mixer(
    ids_hbm, glog_hbm, table_hbm, out_hbm, ids_v, rows_v, outt_v, glog_v, *sems
):
    _sc_body(
        ids_hbm, glog_hbm, table_hbm, out_hbm, ids_v, rows_v, outt_v, glog_v, *sems
    )


def kernel(subject_ids, global_logits, subject_bias):
    out = _mixer(subject_ids, global_logits, subject_bias)
    return out.T
